# baseline (speedup 1.0000x reference)
import functools

import jax
import jax.numpy as jnp
from jax import lax
from jax.experimental import pallas as pl
from jax.experimental.pallas import tpu as pltpu
from jax.experimental.pallas import tpu_sc as plsc

N_ROWS = 100000
EMB = 512

_NC = 2
_NS = 16
_NW = _NC * _NS

ROWS_PER_TILE = 1280
SC_ROWS = ROWS_PER_TILE * _NW
TC_ROWS = N_ROWS - SC_ROWS
TC_BLOCK = 4920
CH = 64
N_CHUNKS = ROWS_PER_TILE // CH
GROUP = 16
SUB = 8

_SC_PARAMS = pltpu.CompilerParams(needs_layout_passes=False)


def _tc_proj_body(t_ref, w_ref, b_ref, out_ref):
    out_ref[...] = (
        jnp.sum(t_ref[...] * w_ref[...], axis=1, keepdims=True) + b_ref[...]
    )


def _tc_projected(table, W, b):
    return pl.pallas_call(
        _tc_proj_body,
        grid=(TC_ROWS // TC_BLOCK,),
        in_specs=[
            pl.BlockSpec((TC_BLOCK, EMB), lambda i: (i, 0)),
            pl.BlockSpec((1, EMB), lambda i: (0, 0)),
            pl.BlockSpec((1, 1), lambda i: (0, 0)),
        ],
        out_specs=pl.BlockSpec((TC_BLOCK, 1), lambda i: (i, 0)),
        out_shape=jax.ShapeDtypeStruct((TC_ROWS, 1), jnp.float32),
    )(table, W, b.reshape(1, 1).astype(jnp.float32))


def _sc_mesh():
    return plsc.VectorSubcoreMesh(core_axis_name="c", subcore_axis_name="s")


@functools.partial(
    pl.kernel,
    mesh=_sc_mesh(),
    compiler_params=_SC_PARAMS,
    out_type=jax.ShapeDtypeStruct((SC_ROWS,), jnp.float32),
    scratch_types=[
        pltpu.VMEM((2, CH, EMB), jnp.float32),
        pltpu.VMEM((EMB,), jnp.float32),
        pltpu.VMEM((16,), jnp.float32),
        pltpu.VMEM((ROWS_PER_TILE,), jnp.float32),
        pltpu.SemaphoreType.DMA,
        pltpu.SemaphoreType.DMA,
    ],
)
def _sc_projected(table_hbm, w_hbm, b_hbm, out_hbm, buf_v, w_v, b_v, out_v,
                  sem0, sem1):
    wid = lax.axis_index("s") * _NC + lax.axis_index("c")
    row0 = TC_ROWS + wid * ROWS_PER_TILE
    pltpu.sync_copy(w_hbm, w_v)
    pltpu.sync_copy(b_hbm, b_v)
    b_vec = b_v[...]
    lane = lax.iota(jnp.int32, 16)
    sems = (sem0, sem1)

    def chunk_copy(c, slot):
        return pltpu.make_async_copy(
            table_hbm.at[pl.ds(row0 + c * CH, CH), :],
            buf_v.at[slot],
            sems[slot],
        )

    def compute_chunk(c, slot):
        @plsc.parallel_loop(0, CH // GROUP, unroll=2)
        def _group(g):
            res = b_vec
            for h in range(GROUP // SUB):
                acc = [jnp.zeros((16,), jnp.float32) for _ in range(SUB)]
                for j in range(EMB // 16):
                    wj = w_v[pl.ds(j * 16, 16)]
                    for r in range(SUB):
                        acc[r] = acc[r] + buf_v[slot,
                                                g * GROUP + h * SUB + r,
                                                pl.ds(j * 16, 16)] * wj
                for r in range(SUB):
                    res = jnp.where(lane == h * SUB + r,
                                    jnp.sum(acc[r]) + b_vec, res)
            out_v[pl.ds(c * CH + g * GROUP, GROUP)] = res

    chunk_copy(0, 0).start()

    @pl.loop(0, N_CHUNKS, step=2)
    def _pair(c):
        chunk_copy(c + 1, 1).start()
        chunk_copy(c, 0).wait()
        compute_chunk(c, 0)

        @pl.when(c + 2 < N_CHUNKS)
        def _():
            chunk_copy(c + 2, 0).start()

        chunk_copy(c + 1, 1).wait()
        compute_chunk(c + 1, 1)

    pltpu.sync_copy(out_v, out_hbm.at[pl.ds(wid * ROWS_PER_TILE,
                                            ROWS_PER_TILE)])


def _make_gather(n_idx):
    assert n_idx % (_NW * 16) == 0
    per_w = n_idx // _NW

    @functools.partial(
        pl.kernel,
        mesh=_sc_mesh(),
        compiler_params=_SC_PARAMS,
        out_type=jax.ShapeDtypeStruct((n_idx,), jnp.float32),
        scratch_types=[
            pltpu.VMEM((N_ROWS,), jnp.float32),
            pltpu.VMEM((per_w,), jnp.int32),
            pltpu.VMEM((per_w,), jnp.float32),
        ],
    )
    def gather_logsoftmax(s_tc_hbm, s_sc_hbm, x_hbm, out_hbm, s_v, idx_v,
                          out_v):
        wid = lax.axis_index("s") * _NC + lax.axis_index("c")
        base = wid * per_w
        pltpu.sync_copy(s_tc_hbm, s_v.at[pl.ds(0, TC_ROWS)])
        pltpu.sync_copy(s_sc_hbm, s_v.at[pl.ds(TC_ROWS, SC_ROWS)])
        pltpu.sync_copy(x_hbm.at[pl.ds(base, per_w)], idx_v)

        @plsc.parallel_loop(0, per_w // 16, unroll=4)
        def _chunk(i):
            sl = pl.ds(i * 16, 16)
            v = plsc.load_gather(s_v, [idx_v[sl]])
            out_v[sl] = v - v
        pltpu.sync_copy(out_v, out_hbm.at[pl.ds(base, per_w)])

    return gather_logsoftmax


def kernel(x, table, W, b):
    B, L = x.shape
    b32 = b.astype(jnp.float32)
    s_tc = _tc_projected(table, W, b32)
    s_sc = _sc_projected(table, W.reshape(EMB),
                         jnp.broadcast_to(b32, (16,)))
    gather = _make_gather(B * L)
    out_flat = gather(s_tc.reshape(TC_ROWS), s_sc,
                      x.reshape(-1).astype(jnp.int32))
    return out_flat.reshape(B, L, 1)


# speedup vs baseline: 1.5248x; 1.5248x over previous
import functools

import jax
import jax.numpy as jnp
from jax import lax
from jax.experimental import pallas as pl
from jax.experimental.pallas import tpu as pltpu
from jax.experimental.pallas import tpu_sc as plsc

N_ROWS = 100000
EMB = 512

_NC = 2
_NS = 16
_NW = _NC * _NS

ROWS_PER_TILE = 1280
SC_ROWS = ROWS_PER_TILE * _NW
TC_ROWS = N_ROWS - SC_ROWS
TC_BLOCK = 4920
CH = 64
N_CHUNKS = ROWS_PER_TILE // CH
GROUP = 16
SUB = 8

_SC_PARAMS = pltpu.CompilerParams(needs_layout_passes=False)


def _tc_proj_body(t_ref, w_ref, b_ref, out_ref):
    out_ref[...] = (
        jnp.sum(t_ref[...] * w_ref[...], axis=1, keepdims=True) + b_ref[...]
    )


def _tc_projected(table, W, b):
    return pl.pallas_call(
        _tc_proj_body,
        grid=(TC_ROWS // TC_BLOCK,),
        in_specs=[
            pl.BlockSpec((TC_BLOCK, EMB), lambda i: (i, 0)),
            pl.BlockSpec((1, EMB), lambda i: (0, 0)),
            pl.BlockSpec((1, 1), lambda i: (0, 0)),
        ],
        out_specs=pl.BlockSpec((TC_BLOCK, 1), lambda i: (i, 0)),
        out_shape=jax.ShapeDtypeStruct((TC_ROWS, 1), jnp.float32),
    )(table, W, b.reshape(1, 1).astype(jnp.float32))


def _sc_mesh():
    return plsc.VectorSubcoreMesh(core_axis_name="c", subcore_axis_name="s")


@functools.partial(
    pl.kernel,
    mesh=_sc_mesh(),
    compiler_params=_SC_PARAMS,
    out_type=jax.ShapeDtypeStruct((SC_ROWS,), jnp.float32),
    scratch_types=[
        pltpu.VMEM((2, CH, EMB), jnp.float32),
        pltpu.VMEM((EMB,), jnp.float32),
        pltpu.VMEM((16,), jnp.float32),
        pltpu.VMEM((ROWS_PER_TILE,), jnp.float32),
        pltpu.VMEM((GROUP, 16), jnp.float32),
        pltpu.SemaphoreType.DMA,
        pltpu.SemaphoreType.DMA,
    ],
)
def _sc_projected(table_hbm, w_hbm, b_hbm, out_hbm, buf_v, w_v, b_v, out_v,
                  part_v, sem0, sem1):
    wid = lax.axis_index("s") * _NC + lax.axis_index("c")
    row0 = TC_ROWS + wid * ROWS_PER_TILE
    pltpu.sync_copy(w_hbm, w_v)
    pltpu.sync_copy(b_hbm, b_v)
    b_vec = b_v[...]
    lane = lax.iota(jnp.int32, 16)
    sems = (sem0, sem1)

    def chunk_copy(c, slot):
        return pltpu.make_async_copy(
            table_hbm.at[pl.ds(row0 + c * CH, CH), :],
            buf_v.at[slot],
            sems[slot],
        )

    def compute_chunk(c, slot):
        @pl.loop(0, CH // GROUP)
        def _group(g):
            for h in range(GROUP // SUB):
                acc = [jnp.zeros((16,), jnp.float32) for _ in range(SUB)]
                for j in range(EMB // 16):
                    wj = w_v[pl.ds(j * 16, 16)]
                    for r in range(SUB):
                        acc[r] = acc[r] + buf_v[slot,
                                                g * GROUP + h * SUB + r,
                                                pl.ds(j * 16, 16)] * wj
                for r in range(SUB):
                    part_v[h * SUB + r, :] = acc[r]
            res = b_vec
            for j in range(16):
                res = res + plsc.load_gather(part_v, [lane, lane * 0 + j])
            out_v[pl.ds(c * CH + g * GROUP, GROUP)] = res

    chunk_copy(0, 0).start()

    @pl.loop(0, N_CHUNKS, step=2)
    def _pair(c):
        chunk_copy(c + 1, 1).start()
        chunk_copy(c, 0).wait()
        compute_chunk(c, 0)

        @pl.when(c + 2 < N_CHUNKS)
        def _():
            chunk_copy(c + 2, 0).start()

        chunk_copy(c + 1, 1).wait()
        compute_chunk(c + 1, 1)

    pltpu.sync_copy(out_v, out_hbm.at[pl.ds(wid * ROWS_PER_TILE,
                                            ROWS_PER_TILE)])


def _make_gather(n_idx):
    assert n_idx % (_NW * 16) == 0
    per_w = n_idx // _NW

    @functools.partial(
        pl.kernel,
        mesh=_sc_mesh(),
        compiler_params=_SC_PARAMS,
        out_type=jax.ShapeDtypeStruct((n_idx,), jnp.float32),
        scratch_types=[
            pltpu.VMEM((N_ROWS,), jnp.float32),
            pltpu.VMEM((per_w,), jnp.int32),
            pltpu.VMEM((per_w,), jnp.float32),
        ],
    )
    def gather_logsoftmax(s_tc_hbm, s_sc_hbm, x_hbm, out_hbm, s_v, idx_v,
                          out_v):
        wid = lax.axis_index("s") * _NC + lax.axis_index("c")
        base = wid * per_w
        pltpu.sync_copy(s_tc_hbm, s_v.at[pl.ds(0, TC_ROWS)])
        pltpu.sync_copy(s_sc_hbm, s_v.at[pl.ds(TC_ROWS, SC_ROWS)])
        pltpu.sync_copy(x_hbm.at[pl.ds(base, per_w)], idx_v)

        def body(i, carry):
            sl = pl.ds(i * 16, 16)
            v = plsc.load_gather(s_v, [idx_v[sl]])
            out_v[sl] = v - v
            return carry

        lax.fori_loop(0, per_w // 16, body, 0)
        pltpu.sync_copy(out_v, out_hbm.at[pl.ds(base, per_w)])

    return gather_logsoftmax


def kernel(x, table, W, b):
    B, L = x.shape
    b32 = b.astype(jnp.float32)
    s_tc = _tc_projected(table, W, b32)
    s_sc = _sc_projected(table, W.reshape(EMB),
                         jnp.broadcast_to(b32, (16,)))
    gather = _make_gather(B * L)
    out_flat = gather(s_tc.reshape(TC_ROWS), s_sc,
                      x.reshape(-1).astype(jnp.int32))
    return out_flat.reshape(B, L, 1)


# speedup vs baseline: 1.8620x; 1.2211x over previous
import functools

import jax
import jax.numpy as jnp
from jax import lax
from jax.experimental import pallas as pl
from jax.experimental.pallas import tpu as pltpu
from jax.experimental.pallas import tpu_sc as plsc

N_ROWS = 100000
EMB = 512

_NC = 2
_NS = 16
_NW = _NC * _NS

ROWS_PER_TILE = 1280
SC_ROWS = ROWS_PER_TILE * _NW
SC_ROW0 = N_ROWS - SC_ROWS
TC_ROWS = 59136
TC_BLOCK = 4224
CH = 64
N_CHUNKS = ROWS_PER_TILE // CH
GROUP = 16
SUB = 8

_SC_PARAMS = pltpu.CompilerParams(needs_layout_passes=False)


def _tc_proj_body(t_ref, w_ref, b_ref, out_ref):
    s = jnp.sum(t_ref[...] * w_ref[...], axis=1)
    out_ref[...] = (s.reshape(TC_BLOCK // 128, 128)
                    + b_ref[...]).reshape(1, TC_BLOCK // 128, 128)


def _tc_projected(table, W, b):
    return pl.pallas_call(
        _tc_proj_body,
        grid=(TC_ROWS // TC_BLOCK,),
        in_specs=[
            pl.BlockSpec((TC_BLOCK, EMB), lambda i: (i, 0)),
            pl.BlockSpec((1, EMB), lambda i: (0, 0)),
            pl.BlockSpec((1, 1), lambda i: (0, 0)),
        ],
        out_specs=pl.BlockSpec((1, TC_BLOCK // 128, 128),
                               lambda i: (i, 0, 0)),
        out_shape=jax.ShapeDtypeStruct(
            (TC_ROWS // TC_BLOCK, TC_BLOCK // 128, 128), jnp.float32),
    )(table, W, b.reshape(1, 1).astype(jnp.float32))


def _sc_mesh():
    return plsc.VectorSubcoreMesh(core_axis_name="c", subcore_axis_name="s")


@functools.partial(
    pl.kernel,
    mesh=_sc_mesh(),
    compiler_params=_SC_PARAMS,
    out_type=jax.ShapeDtypeStruct((SC_ROWS,), jnp.float32),
    scratch_types=[
        pltpu.VMEM((2, CH, EMB), jnp.float32),
        pltpu.VMEM((EMB,), jnp.float32),
        pltpu.VMEM((16,), jnp.float32),
        pltpu.VMEM((ROWS_PER_TILE,), jnp.float32),
        pltpu.VMEM((GROUP, 16), jnp.float32),
        pltpu.SemaphoreType.DMA,
        pltpu.SemaphoreType.DMA,
    ],
)
def _sc_projected(table_hbm, w_hbm, b_hbm, out_hbm, buf_v, w_v, b_v, out_v,
                  part_v, sem0, sem1):
    wid = lax.axis_index("s") * _NC + lax.axis_index("c")
    row0 = SC_ROW0 + wid * ROWS_PER_TILE
    pltpu.sync_copy(w_hbm, w_v)
    pltpu.sync_copy(b_hbm, b_v)
    b_vec = b_v[...]
    lane = lax.iota(jnp.int32, 16)
    sems = (sem0, sem1)

    def chunk_copy(c, slot):
        return pltpu.make_async_copy(
            table_hbm.at[pl.ds(row0 + c * CH, CH), :],
            buf_v.at[slot],
            sems[slot],
        )

    def compute_chunk(c, slot):
        @pl.loop(0, CH // GROUP)
        def _group(g):
            for h in range(GROUP // SUB):
                acc = [jnp.zeros((16,), jnp.float32) for _ in range(SUB)]
                for j in range(EMB // 16):
                    wj = w_v[pl.ds(j * 16, 16)]
                    for r in range(SUB):
                        acc[r] = acc[r] + buf_v[slot,
                                                g * GROUP + h * SUB + r,
                                                pl.ds(j * 16, 16)] * wj
                for r in range(SUB):
                    part_v[h * SUB + r, :] = acc[r]
            res = b_vec
            for j in range(16):
                res = res + plsc.load_gather(part_v, [lane, lane * 0 + j])
            out_v[pl.ds(c * CH + g * GROUP, GROUP)] = res

    chunk_copy(0, 0).start()

    @pl.loop(0, N_CHUNKS, step=2)
    def _pair(c):
        chunk_copy(c + 1, 1).start()
        chunk_copy(c, 0).wait()
        compute_chunk(c, 0)

        @pl.when(c + 2 < N_CHUNKS)
        def _():
            chunk_copy(c + 2, 0).start()

        chunk_copy(c + 1, 1).wait()
        compute_chunk(c + 1, 1)

    pltpu.sync_copy(out_v, out_hbm.at[pl.ds(wid * ROWS_PER_TILE,
                                            ROWS_PER_TILE)])


def _make_gather(n_idx):
    assert n_idx % (_NW * 16) == 0
    per_w = n_idx // _NW

    @functools.partial(
        pl.kernel,
        mesh=_sc_mesh(),
        compiler_params=_SC_PARAMS,
        out_type=jax.ShapeDtypeStruct((n_idx,), jnp.float32),
        scratch_types=[
            pltpu.VMEM((N_ROWS,), jnp.float32),
            pltpu.VMEM((per_w,), jnp.int32),
            pltpu.VMEM((per_w,), jnp.float32),
        ],
    )
    def gather_logsoftmax(s_tc_hbm, s_sc_hbm, x_hbm, out_hbm, s_v, idx_v,
                          out_v):
        wid = lax.axis_index("s") * _NC + lax.axis_index("c")
        base = wid * per_w
        pltpu.sync_copy(s_tc_hbm, s_v.at[pl.ds(0, TC_ROWS)])
        pltpu.sync_copy(s_sc_hbm.at[pl.ds(TC_ROWS - SC_ROW0,
                                          N_ROWS - TC_ROWS)],
                        s_v.at[pl.ds(TC_ROWS, N_ROWS - TC_ROWS)])
        pltpu.sync_copy(x_hbm.at[pl.ds(base, per_w)], idx_v)

        def body(i, carry):
            sl = pl.ds(i * 16, 16)
            v = plsc.load_gather(s_v, [idx_v[sl]])
            out_v[sl] = v - v
            return carry

        lax.fori_loop(0, per_w // 16, body, 0)
        pltpu.sync_copy(out_v, out_hbm.at[pl.ds(base, per_w)])

    return gather_logsoftmax


def kernel(x, table, W, b):
    B, L = x.shape
    b32 = b.astype(jnp.float32)
    s_tc = _tc_projected(table, W, b32)
    s_sc = _sc_projected(table, W.reshape(EMB),
                         jnp.broadcast_to(b32, (16,)))
    gather = _make_gather(B * L)
    out_flat = gather(s_tc.reshape(TC_ROWS), s_sc,
                      x.reshape(-1).astype(jnp.int32))
    return out_flat.reshape(B, L, 1)


# speedup vs baseline: 1.9206x; 1.0315x over previous
import functools

import jax
import jax.numpy as jnp
from jax import lax
from jax.experimental import pallas as pl
from jax.experimental.pallas import tpu as pltpu
from jax.experimental.pallas import tpu_sc as plsc

N_ROWS = 100000
EMB = 512

_NC = 2
_NS = 16
_NW = _NC * _NS

ROWS_PER_TILE = 1280
SC_ROWS = ROWS_PER_TILE * _NW
SC_ROW0 = N_ROWS - SC_ROWS
TC_ROWS = 100096
TC_BLOCK = 4352
CH = 64
N_CHUNKS = ROWS_PER_TILE // CH
GROUP = 16
SUB = 8

_SC_PARAMS = pltpu.CompilerParams(needs_layout_passes=False)


def _tc_proj_body(t_ref, w_ref, b_ref, out_ref):
    s = jnp.sum(t_ref[...] * w_ref[...], axis=1)
    out_ref[...] = (s.reshape(TC_BLOCK // 128, 128)
                    + b_ref[...]).reshape(1, TC_BLOCK // 128, 128)


def _tc_projected(table, W, b):
    return pl.pallas_call(
        _tc_proj_body,
        grid=(TC_ROWS // TC_BLOCK,),
        in_specs=[
            pl.BlockSpec((TC_BLOCK, EMB), lambda i: (i, 0)),
            pl.BlockSpec((1, EMB), lambda i: (0, 0)),
            pl.BlockSpec((1, 1), lambda i: (0, 0)),
        ],
        out_specs=pl.BlockSpec((1, TC_BLOCK // 128, 128),
                               lambda i: (i, 0, 0)),
        out_shape=jax.ShapeDtypeStruct(
            (TC_ROWS // TC_BLOCK, TC_BLOCK // 128, 128), jnp.float32),
    )(table, W, b.reshape(1, 1).astype(jnp.float32))


def _sc_mesh():
    return plsc.VectorSubcoreMesh(core_axis_name="c", subcore_axis_name="s")


@functools.partial(
    pl.kernel,
    mesh=_sc_mesh(),
    compiler_params=_SC_PARAMS,
    out_type=jax.ShapeDtypeStruct((SC_ROWS,), jnp.float32),
    scratch_types=[
        pltpu.VMEM((2, CH, EMB), jnp.float32),
        pltpu.VMEM((EMB,), jnp.float32),
        pltpu.VMEM((16,), jnp.float32),
        pltpu.VMEM((ROWS_PER_TILE,), jnp.float32),
        pltpu.VMEM((GROUP, 16), jnp.float32),
        pltpu.SemaphoreType.DMA,
        pltpu.SemaphoreType.DMA,
    ],
)
def _sc_projected(table_hbm, w_hbm, b_hbm, out_hbm, buf_v, w_v, b_v, out_v,
                  part_v, sem0, sem1):
    wid = lax.axis_index("s") * _NC + lax.axis_index("c")
    row0 = SC_ROW0 + wid * ROWS_PER_TILE
    pltpu.sync_copy(w_hbm, w_v)
    pltpu.sync_copy(b_hbm, b_v)
    b_vec = b_v[...]
    lane = lax.iota(jnp.int32, 16)
    sems = (sem0, sem1)

    def chunk_copy(c, slot):
        return pltpu.make_async_copy(
            table_hbm.at[pl.ds(row0 + c * CH, CH), :],
            buf_v.at[slot],
            sems[slot],
        )

    def compute_chunk(c, slot):
        @pl.loop(0, CH // GROUP)
        def _group(g):
            for h in range(GROUP // SUB):
                acc = [jnp.zeros((16,), jnp.float32) for _ in range(SUB)]
                for j in range(EMB // 16):
                    wj = w_v[pl.ds(j * 16, 16)]
                    for r in range(SUB):
                        acc[r] = acc[r] + buf_v[slot,
                                                g * GROUP + h * SUB + r,
                                                pl.ds(j * 16, 16)] * wj
                for r in range(SUB):
                    part_v[h * SUB + r, :] = acc[r]
            res = b_vec
            for j in range(16):
                res = res + plsc.load_gather(part_v, [lane, lane * 0 + j])
            out_v[pl.ds(c * CH + g * GROUP, GROUP)] = res

    chunk_copy(0, 0).start()

    @pl.loop(0, N_CHUNKS, step=2)
    def _pair(c):
        chunk_copy(c + 1, 1).start()
        chunk_copy(c, 0).wait()
        compute_chunk(c, 0)

        @pl.when(c + 2 < N_CHUNKS)
        def _():
            chunk_copy(c + 2, 0).start()

        chunk_copy(c + 1, 1).wait()
        compute_chunk(c + 1, 1)

    pltpu.sync_copy(out_v, out_hbm.at[pl.ds(wid * ROWS_PER_TILE,
                                            ROWS_PER_TILE)])


def _make_gather(n_idx):
    assert n_idx % (_NW * 16) == 0
    per_w = n_idx // _NW

    @functools.partial(
        pl.kernel,
        mesh=_sc_mesh(),
        compiler_params=_SC_PARAMS,
        out_type=jax.ShapeDtypeStruct((n_idx,), jnp.float32),
        scratch_types=[
            pltpu.VMEM((N_ROWS,), jnp.float32),
            pltpu.VMEM((per_w,), jnp.int32),
            pltpu.VMEM((per_w,), jnp.float32),
        ],
    )
    def gather_logsoftmax(s_tc_hbm, x_hbm, out_hbm, s_v, idx_v, out_v):
        wid = lax.axis_index("s") * _NC + lax.axis_index("c")
        base = wid * per_w
        pltpu.sync_copy(s_tc_hbm.at[pl.ds(0, N_ROWS)],
                        s_v.at[pl.ds(0, N_ROWS)])
        pltpu.sync_copy(x_hbm.at[pl.ds(base, per_w)], idx_v)

        def body(i, carry):
            sl = pl.ds(i * 16, 16)
            v = plsc.load_gather(s_v, [idx_v[sl]])
            out_v[sl] = v - v
            return carry

        lax.fori_loop(0, per_w // 16, body, 0)
        pltpu.sync_copy(out_v, out_hbm.at[pl.ds(base, per_w)])

    return gather_logsoftmax


def kernel(x, table, W, b):
    B, L = x.shape
    b32 = b.astype(jnp.float32)
    s_tc = _tc_projected(table, W, b32)
    gather = _make_gather(B * L)
    out_flat = gather(s_tc.reshape(TC_ROWS),
                      x.reshape(-1).astype(jnp.int32))
    return out_flat.reshape(B, L, 1)
